# Initial kernel scaffold; baseline (speedup 1.0000x reference)
import jax
import jax.numpy as jnp
from jax.experimental import pallas as pl


def kernel(h_seq, entity_keys, Wi, bi, W_ih, W_hh, b_ih, b_hh, e0):
    raise NotImplementedError("write your pallas kernel here")


# speedup vs baseline: 3.8532x; 3.8532x over previous
import jax
import jax.numpy as jnp
from jax.experimental import pallas as pl
from jax.experimental.pallas import tpu as pltpu

B, T, D = 16, 2048, 1024
N_E, D_E = 8, 64
BH = 8
TBLK = 128
NT = T // TBLK
G3 = 3 * D_E


def _entity_kernel(h_ref, mc_ref, c_ref, bih_ref, bhh_ref, whh_ref, e0_ref,
                   out_ref, state_ref, gxb_ref):
    j = pl.program_id(1)

    x2 = h_ref[...].reshape(BH * TBLK, D)
    mm = jnp.dot(x2, mc_ref[...], preferred_element_type=jnp.float32)
    pre2 = mm[:, :G3] + c_ref[...]
    lg = mm[:, G3:G3 + N_E]
    m = jnp.max(lg, axis=-1, keepdims=True)
    p = jnp.exp(lg - m)
    w2 = p / jnp.sum(p, axis=-1, keepdims=True)

    pre3 = pre2.reshape(BH, TBLK, G3)
    w3 = w2.reshape(BH, TBLK, N_E)
    gxb4 = w3[..., None] * pre3[:, :, None, :] + bih_ref[...]
    gxb_ref[...] = jnp.transpose(gxb4, (1, 0, 2, 3)).reshape(TBLK, BH * N_E, G3)

    @pl.when(j == 0)
    def _():
        state_ref[...] = jnp.concatenate([e0_ref[...]] * BH, axis=0)

    def body(t, st):
        gx = gxb_ref[t]
        gh = jnp.dot(st, whh_ref[...],
                     preferred_element_type=jnp.float32) + bhh_ref[...]
        rz = jax.nn.sigmoid(gx[:, :2 * D_E] + gh[:, :2 * D_E])
        r = rz[:, :D_E]
        z = rz[:, D_E:2 * D_E]
        n = jnp.tanh(gx[:, 2 * D_E:] + r * gh[:, 2 * D_E:])
        new = n + z * (st - n)
        out_ref[:, pl.ds(t, 1), :, :] = new.reshape(BH, 1, N_E, D_E)
        return new

    st = jax.lax.fori_loop(0, TBLK, body, state_ref[...], unroll=2)
    state_ref[...] = st


def kernel(h_seq, entity_keys, Wi, bi, W_ih, W_hh, b_ih, b_hh, e0):
    m_pre = (W_ih @ Wi).T
    keys_t = entity_keys.T / jnp.sqrt(jnp.float32(D))
    mc = jnp.concatenate([m_pre, keys_t], axis=1)
    c = (W_ih @ bi).reshape(1, G3)
    bih2 = b_ih.reshape(1, G3)
    bhh2 = b_hh.reshape(1, G3)
    whh_t = W_hh.T

    stack = pl.pallas_call(
        _entity_kernel,
        grid=(2, NT),
        in_specs=[
            pl.BlockSpec((BH, TBLK, D), lambda i, j: (i, j, 0)),
            pl.BlockSpec((D, G3 + N_E), lambda i, j: (0, 0)),
            pl.BlockSpec((1, G3), lambda i, j: (0, 0)),
            pl.BlockSpec((1, G3), lambda i, j: (0, 0)),
            pl.BlockSpec((1, G3), lambda i, j: (0, 0)),
            pl.BlockSpec((D_E, G3), lambda i, j: (0, 0)),
            pl.BlockSpec((N_E, D_E), lambda i, j: (0, 0)),
        ],
        out_specs=pl.BlockSpec((BH, TBLK, N_E, D_E), lambda i, j: (i, j, 0, 0)),
        out_shape=jax.ShapeDtypeStruct((B, T, N_E, D_E), jnp.float32),
        scratch_shapes=[
            pltpu.VMEM((BH * N_E, D_E), jnp.float32),
            pltpu.VMEM((TBLK, BH * N_E, G3), jnp.float32),
        ],
        compiler_params=pltpu.CompilerParams(
            dimension_semantics=("parallel", "arbitrary"),
            vmem_limit_bytes=100 * 1024 * 1024,
        ),
    )(h_seq, mc, c, bih2, bhh2, whh_t, e0)

    entity_seq = stack.reshape(B, T, N_E * D_E)
    return entity_seq, stack
